# baseline (speedup 1.0000x reference)
import functools

import jax
import jax.numpy as jnp
from jax import lax
from jax.experimental import pallas as pl
from jax.experimental.pallas import tpu as pltpu
from jax.experimental.pallas import tpu_sc as plsc


def _topk_body(x1_ref, x2t_ref, idx_ref, w_ref, *, S):
    b = pl.program_id(0)
    x1 = x1_ref[0]
    x2 = x2t_ref[0]
    NT = x1.shape[0]
    cross = lax.dot_general(x1, x2, (((1,), (0,)), ((), ())),
                            precision=lax.Precision.HIGHEST,
                            preferred_element_type=jnp.float32)
    n1 = jnp.sum(x1 * x1, axis=1, keepdims=True)
    n2 = jnp.sum(x2 * x2, axis=0, keepdims=True)
    d = jnp.maximum(n1 - 2.0 * cross + n2, 0.0)
    iota_f = lax.broadcasted_iota(jnp.int32, (NT, S), 1).astype(jnp.float32)
    idxs, ws = [], []
    for k in range(3):
        m = jnp.min(d, axis=1, keepdims=True)
        ikf = jnp.min(jnp.where(d == m, iota_f, jnp.float32(S)),
                      axis=1, keepdims=True)
        idxs.append(ikf.astype(jnp.int32))
        ws.append(1.0 / (m + 1e-8))
        if k < 2:
            d = jnp.where(iota_f == ikf, jnp.float32(jnp.inf), d)
    wcat = jnp.concatenate(ws, axis=1)
    wcat = wcat / jnp.sum(wcat, axis=1, keepdims=True)
    icat = jnp.concatenate(idxs, axis=1) + b * S
    idx_ref[0] = icat
    w_ref[0] = wcat


def _topk(xyz1, xyz2t, NT):
    B, N, _ = xyz1.shape
    S = xyz2t.shape[2]
    grid = (B, N // NT)
    return pl.pallas_call(
        functools.partial(_topk_body, S=S),
        grid=grid,
        in_specs=[
            pl.BlockSpec((1, NT, 3), lambda b, i: (b, i, 0)),
            pl.BlockSpec((1, 3, S), lambda b, i: (b, 0, 0)),
        ],
        out_specs=[
            pl.BlockSpec((1, NT, 3), lambda b, i: (b, i, 0)),
            pl.BlockSpec((1, NT, 3), lambda b, i: (b, i, 0)),
        ],
        out_shape=[
            jax.ShapeDtypeStruct((B, N, 3), jnp.int32),
            jax.ShapeDtypeStruct((B, N, 3), jnp.float32),
        ],
    )(xyz1, xyz2t)


def _sc_gather(table, idx, CH=128):
    M, = idx.shape
    R, C = table.shape
    info = plsc.get_sparse_core_info()
    NW = info.num_cores * info.num_subcores
    n_ch = M // (NW * CH)
    idx2 = idx.reshape(M // CH, CH)
    mesh = plsc.VectorSubcoreMesh(core_axis_name="c", subcore_axis_name="s")

    @functools.partial(
        pl.kernel,
        mesh=mesh,
        out_type=jax.ShapeDtypeStruct((M // CH, CH, C), jnp.float32),
        scratch_types=[
            pltpu.VMEM((CH,), jnp.int32),
            pltpu.VMEM((CH, C), jnp.float32),
            pltpu.SemaphoreType.DMA,
        ],
    )
    def gather_k(idx_hbm, table_hbm, out_hbm, idx_v, rows_v, sem):
        wid = lax.axis_index("c") * info.num_subcores + lax.axis_index("s")

        def body(j, carry):
            blk = wid * n_ch + j
            pltpu.sync_copy(idx_hbm.at[blk], idx_v)
            pltpu.async_copy(table_hbm.at[idx_v], rows_v, sem).wait()
            pltpu.sync_copy(rows_v, out_hbm.at[blk])
            return carry

        lax.fori_loop(0, n_ch, body, 0)

    return gather_k(idx2, table).reshape(M, C)


def _mlp1_body(g_ref, w_ref, p1_ref, W1t_ref, b1_ref, y_ref, s_ref):
    first = (pl.program_id(0) == 0) & (pl.program_id(1) == 0)
    g = g_ref[...]
    w = w_ref[0]
    interp = (g[0, 0] * w[:, 0:1] + g[1, 0] * w[:, 1:2] + g[2, 0] * w[:, 2:3])
    x = jnp.concatenate([p1_ref[0], interp], axis=1)
    y = jnp.dot(x, W1t_ref[...], preferred_element_type=jnp.float32)
    y = y + b1_ref[...]
    y_ref[0] = y
    acc = jnp.concatenate([jnp.sum(y, axis=0, keepdims=True),
                           jnp.sum(y * y, axis=0, keepdims=True)], axis=0)

    @pl.when(first)
    def _():
        s_ref[...] = jnp.zeros_like(s_ref)

    s_ref[...] += acc


def _mlp1(g4, w3, p1t, W1t, b1row, NT):
    _, B, N, C2 = g4.shape
    C1 = p1t.shape[2]
    Co = W1t.shape[1]
    grid = (B, N // NT)
    return pl.pallas_call(
        _mlp1_body,
        grid=grid,
        in_specs=[
            pl.BlockSpec((3, 1, NT, C2), lambda b, i: (0, b, i, 0)),
            pl.BlockSpec((1, NT, 3), lambda b, i: (b, i, 0)),
            pl.BlockSpec((1, NT, C1), lambda b, i: (b, i, 0)),
            pl.BlockSpec((C1 + C2, Co), lambda b, i: (0, 0)),
            pl.BlockSpec((1, Co), lambda b, i: (0, 0)),
        ],
        out_specs=[
            pl.BlockSpec((1, NT, Co), lambda b, i: (b, i, 0)),
            pl.BlockSpec((2, Co), lambda b, i: (0, 0)),
        ],
        out_shape=[
            jax.ShapeDtypeStruct((B, N, Co), jnp.float32),
            jax.ShapeDtypeStruct((2, Co), jnp.float32),
        ],
    )(g4, w3, p1t, W1t, b1row)


def _mlp2_body(y1_ref, sc1_ref, W2t_ref, b2_ref, y2_ref, s_ref):
    first = (pl.program_id(0) == 0) & (pl.program_id(1) == 0)
    sc = sc1_ref[...]
    z = jnp.maximum(y1_ref[0] * sc[0:1, :] + sc[1:2, :], 0.0)
    y2 = jnp.dot(z, W2t_ref[...], preferred_element_type=jnp.float32)
    y2 = y2 + b2_ref[...]
    y2_ref[0] = y2
    acc = jnp.concatenate([jnp.sum(y2, axis=0, keepdims=True),
                           jnp.sum(y2 * y2, axis=0, keepdims=True)], axis=0)

    @pl.when(first)
    def _():
        s_ref[...] = jnp.zeros_like(s_ref)

    s_ref[...] += acc


def _mlp2(y1, sc1, W2t, b2row, NT):
    B, N, Ci = y1.shape
    Co = W2t.shape[1]
    grid = (B, N // NT)
    return pl.pallas_call(
        _mlp2_body,
        grid=grid,
        in_specs=[
            pl.BlockSpec((1, NT, Ci), lambda b, i: (b, i, 0)),
            pl.BlockSpec((2, Ci), lambda b, i: (0, 0)),
            pl.BlockSpec((Ci, Co), lambda b, i: (0, 0)),
            pl.BlockSpec((1, Co), lambda b, i: (0, 0)),
        ],
        out_specs=[
            pl.BlockSpec((1, NT, Co), lambda b, i: (b, i, 0)),
            pl.BlockSpec((2, Co), lambda b, i: (0, 0)),
        ],
        out_shape=[
            jax.ShapeDtypeStruct((B, N, Co), jnp.float32),
            jax.ShapeDtypeStruct((2, Co), jnp.float32),
        ],
    )(y1, sc1, W2t, b2row)


def _bn2_body(y2_ref, sc2_ref, o_ref):
    sc = sc2_ref[...]
    o_ref[0] = jnp.maximum(y2_ref[0] * sc[0:1, :] + sc[1:2, :], 0.0)


def _bn2(y2, sc2, NT):
    B, N, Co = y2.shape
    grid = (B, N // NT)
    return pl.pallas_call(
        _bn2_body,
        grid=grid,
        in_specs=[
            pl.BlockSpec((1, NT, Co), lambda b, i: (b, i, 0)),
            pl.BlockSpec((2, Co), lambda b, i: (0, 0)),
        ],
        out_specs=pl.BlockSpec((1, NT, Co), lambda b, i: (b, i, 0)),
        out_shape=jax.ShapeDtypeStruct((B, N, Co), jnp.float32),
    )(y2, sc2)


def _bn_scale_shift(sums, count, g, be):
    mean = sums[0] / count
    var = sums[1] / count - mean * mean
    scale = g * lax.rsqrt(var + 1e-5)
    shift = be - mean * scale
    return jnp.stack([scale, shift])


def kernel(xyz1, xyz2, points1, points2, W1, b1, g1, be1, W2, b2, g2, be2):
    B, N, _ = xyz1.shape
    S = xyz2.shape[1]
    C1 = points1.shape[1]
    C2 = points2.shape[1]
    NT = 512

    xyz2t = jnp.transpose(xyz2, (0, 2, 1))
    idx3, w3 = _topk(xyz1, xyz2t, NT)

    pts2 = jnp.transpose(points2, (0, 2, 1)).reshape(B * S, C2)
    idx_flat = jnp.transpose(idx3, (2, 0, 1)).reshape(-1)
    gathered = _sc_gather(pts2, idx_flat)
    g4 = gathered.reshape(3, B, N, C2)

    p1t = jnp.transpose(points1, (0, 2, 1))
    y1, s1 = _mlp1(g4, w3, p1t, jnp.transpose(W1), b1[None, :], NT)
    sc1 = _bn_scale_shift(s1, B * N, g1, be1)
    y2, s2 = _mlp2(y1, sc1, jnp.transpose(W2), b2[None, :], NT)
    sc2 = _bn_scale_shift(s2, B * N, g2, be2)
    outt = _bn2(y2, sc2, NT)
    return jnp.transpose(outt, (0, 2, 1))


# speedup vs baseline: 1.1409x; 1.1409x over previous
import functools

import jax
import jax.numpy as jnp
from jax import lax
from jax.experimental import pallas as pl
from jax.experimental.pallas import tpu as pltpu
from jax.experimental.pallas import tpu_sc as plsc


def _topk_body(x1_ref, x2t_ref, idx_ref, w_ref, *, S):
    b = pl.program_id(0)
    x1 = x1_ref[0]
    x2 = x2t_ref[0]
    NT = x1.shape[0]
    d = None
    for c in range(3):
        diff = x1[:, c:c + 1] - x2[c:c + 1, :]
        sq = diff * diff
        d = sq if d is None else d + sq
    iota_f = lax.broadcasted_iota(jnp.int32, (NT, S), 1).astype(jnp.float32)
    idxs, ws = [], []
    for k in range(3):
        m = jnp.min(d, axis=1, keepdims=True)
        ikf = jnp.min(jnp.where(d == m, iota_f, jnp.float32(S)),
                      axis=1, keepdims=True)
        idxs.append(ikf.astype(jnp.int32))
        ws.append(1.0 / (m + 1e-8))
        if k < 2:
            d = jnp.where(iota_f == ikf, jnp.float32(jnp.inf), d)
    wcat = jnp.concatenate(ws, axis=1)
    wcat = wcat / jnp.sum(wcat, axis=1, keepdims=True)
    icat = jnp.concatenate(idxs, axis=1) + b * S
    idx_ref[0] = icat
    w_ref[0] = wcat


def _topk(xyz1, xyz2t, NT):
    B, N, _ = xyz1.shape
    S = xyz2t.shape[2]
    grid = (B, N // NT)
    return pl.pallas_call(
        functools.partial(_topk_body, S=S),
        grid=grid,
        in_specs=[
            pl.BlockSpec((1, NT, 3), lambda b, i: (b, i, 0)),
            pl.BlockSpec((1, 3, S), lambda b, i: (b, 0, 0)),
        ],
        out_specs=[
            pl.BlockSpec((1, NT, 3), lambda b, i: (b, i, 0)),
            pl.BlockSpec((1, NT, 3), lambda b, i: (b, i, 0)),
        ],
        out_shape=[
            jax.ShapeDtypeStruct((B, N, 3), jnp.int32),
            jax.ShapeDtypeStruct((B, N, 3), jnp.float32),
        ],
    )(xyz1, xyz2t)


def _sc_gather(table, idx, CH=128):
    M, = idx.shape
    R, C = table.shape
    info = plsc.get_sparse_core_info()
    NW = info.num_cores * info.num_subcores
    n_ch = M // (NW * CH)
    idx2 = idx.reshape(M // CH, CH)
    mesh = plsc.VectorSubcoreMesh(core_axis_name="c", subcore_axis_name="s")

    @functools.partial(
        pl.kernel,
        mesh=mesh,
        out_type=jax.ShapeDtypeStruct((M // CH, CH, C), jnp.float32),
        scratch_types=[
            pltpu.VMEM((CH,), jnp.int32),
            pltpu.VMEM((CH, C), jnp.float32),
            pltpu.SemaphoreType.DMA,
        ],
    )
    def gather_k(idx_hbm, table_hbm, out_hbm, idx_v, rows_v, sem):
        wid = lax.axis_index("c") * info.num_subcores + lax.axis_index("s")

        def body(j, carry):
            blk = wid * n_ch + j
            pltpu.sync_copy(idx_hbm.at[blk], idx_v)
            pltpu.async_copy(table_hbm.at[idx_v], rows_v, sem).wait()
            pltpu.sync_copy(rows_v, out_hbm.at[blk])
            return carry

        lax.fori_loop(0, n_ch, body, 0)

    return gather_k(idx2, table).reshape(M, C)


def _mlp1_body(g_ref, w_ref, p1_ref, W1t_ref, b1_ref, y_ref, s_ref):
    first = (pl.program_id(0) == 0) & (pl.program_id(1) == 0)
    g = g_ref[...]
    w = w_ref[0]
    interp = (g[0, 0] * w[:, 0:1] + g[1, 0] * w[:, 1:2] + g[2, 0] * w[:, 2:3])
    x = jnp.concatenate([p1_ref[0], interp], axis=1)
    y = jnp.dot(x, W1t_ref[...], preferred_element_type=jnp.float32)
    y = y + b1_ref[...]
    y_ref[0] = y
    acc = jnp.concatenate([jnp.sum(y, axis=0, keepdims=True),
                           jnp.sum(y * y, axis=0, keepdims=True)], axis=0)

    @pl.when(first)
    def _():
        s_ref[...] = jnp.zeros_like(s_ref)

    s_ref[...] += acc


def _mlp1(g4, w3, p1t, W1t, b1row, NT):
    _, B, N, C2 = g4.shape
    C1 = p1t.shape[2]
    Co = W1t.shape[1]
    grid = (B, N // NT)
    return pl.pallas_call(
        _mlp1_body,
        grid=grid,
        in_specs=[
            pl.BlockSpec((3, 1, NT, C2), lambda b, i: (0, b, i, 0)),
            pl.BlockSpec((1, NT, 3), lambda b, i: (b, i, 0)),
            pl.BlockSpec((1, NT, C1), lambda b, i: (b, i, 0)),
            pl.BlockSpec((C1 + C2, Co), lambda b, i: (0, 0)),
            pl.BlockSpec((1, Co), lambda b, i: (0, 0)),
        ],
        out_specs=[
            pl.BlockSpec((1, NT, Co), lambda b, i: (b, i, 0)),
            pl.BlockSpec((2, Co), lambda b, i: (0, 0)),
        ],
        out_shape=[
            jax.ShapeDtypeStruct((B, N, Co), jnp.float32),
            jax.ShapeDtypeStruct((2, Co), jnp.float32),
        ],
    )(g4, w3, p1t, W1t, b1row)


def _mlp2_body(y1_ref, sc1_ref, W2t_ref, b2_ref, y2_ref, s_ref):
    first = (pl.program_id(0) == 0) & (pl.program_id(1) == 0)
    sc = sc1_ref[...]
    z = jnp.maximum(y1_ref[0] * sc[0:1, :] + sc[1:2, :], 0.0)
    y2 = jnp.dot(z, W2t_ref[...], preferred_element_type=jnp.float32)
    y2 = y2 + b2_ref[...]
    y2_ref[0] = y2
    acc = jnp.concatenate([jnp.sum(y2, axis=0, keepdims=True),
                           jnp.sum(y2 * y2, axis=0, keepdims=True)], axis=0)

    @pl.when(first)
    def _():
        s_ref[...] = jnp.zeros_like(s_ref)

    s_ref[...] += acc


def _mlp2(y1, sc1, W2t, b2row, NT):
    B, N, Ci = y1.shape
    Co = W2t.shape[1]
    grid = (B, N // NT)
    return pl.pallas_call(
        _mlp2_body,
        grid=grid,
        in_specs=[
            pl.BlockSpec((1, NT, Ci), lambda b, i: (b, i, 0)),
            pl.BlockSpec((2, Ci), lambda b, i: (0, 0)),
            pl.BlockSpec((Ci, Co), lambda b, i: (0, 0)),
            pl.BlockSpec((1, Co), lambda b, i: (0, 0)),
        ],
        out_specs=[
            pl.BlockSpec((1, NT, Co), lambda b, i: (b, i, 0)),
            pl.BlockSpec((2, Co), lambda b, i: (0, 0)),
        ],
        out_shape=[
            jax.ShapeDtypeStruct((B, N, Co), jnp.float32),
            jax.ShapeDtypeStruct((2, Co), jnp.float32),
        ],
    )(y1, sc1, W2t, b2row)


def _bn2_body(y2_ref, sc2_ref, o_ref):
    sc = sc2_ref[...]
    o_ref[0] = jnp.maximum(y2_ref[0] * sc[0:1, :] + sc[1:2, :], 0.0)


def _bn2(y2, sc2, NT):
    B, N, Co = y2.shape
    grid = (B, N // NT)
    return pl.pallas_call(
        _bn2_body,
        grid=grid,
        in_specs=[
            pl.BlockSpec((1, NT, Co), lambda b, i: (b, i, 0)),
            pl.BlockSpec((2, Co), lambda b, i: (0, 0)),
        ],
        out_specs=pl.BlockSpec((1, NT, Co), lambda b, i: (b, i, 0)),
        out_shape=jax.ShapeDtypeStruct((B, N, Co), jnp.float32),
    )(y2, sc2)


def _bn_scale_shift(sums, count, g, be):
    mean = sums[0] / count
    var = sums[1] / count - mean * mean
    scale = g * lax.rsqrt(var + 1e-5)
    shift = be - mean * scale
    return jnp.stack([scale, shift])


def kernel(xyz1, xyz2, points1, points2, W1, b1, g1, be1, W2, b2, g2, be2):
    B, N, _ = xyz1.shape
    S = xyz2.shape[1]
    C1 = points1.shape[1]
    C2 = points2.shape[1]
    NT = 512

    xyz2t = jnp.transpose(xyz2, (0, 2, 1))
    idx3, w3 = _topk(xyz1, xyz2t, NT)

    pts2 = jnp.transpose(points2, (0, 2, 1)).reshape(B * S, C2)
    idx_flat = jnp.transpose(idx3, (2, 0, 1)).reshape(-1)
    gathered = _sc_gather(pts2, idx_flat)
    g4 = gathered.reshape(3, B, N, C2)

    p1t = jnp.transpose(points1, (0, 2, 1))
    y1, s1 = _mlp1(g4, w3, p1t, jnp.transpose(W1), b1[None, :], NT)
    sc1 = _bn_scale_shift(s1, B * N, g1, be1)
    y2, s2 = _mlp2(y1, sc1, jnp.transpose(W2), b2[None, :], NT)
    sc2 = _bn_scale_shift(s2, B * N, g2, be2)
    outt = _bn2(y2, sc2, NT)
    return jnp.transpose(outt, (0, 2, 1))


# speedup vs baseline: 1.2048x; 1.0560x over previous
import functools

import jax
import jax.numpy as jnp
from jax import lax
from jax.experimental import pallas as pl
from jax.experimental.pallas import tpu as pltpu
from jax.experimental.pallas import tpu_sc as plsc


def _topk_body(x1_ref, x2t_ref, idx_ref, w_ref, *, S):
    b = pl.program_id(0)
    x1 = x1_ref[0]
    x2 = x2t_ref[0]
    NT = x1.shape[0]
    d = None
    for c in range(3):
        diff = x1[:, c:c + 1] - x2[c:c + 1, :]
        sq = diff * diff
        d = sq if d is None else d + sq
    iota_f = lax.broadcasted_iota(jnp.int32, (NT, S), 1).astype(jnp.float32)
    idxs, ws = [], []
    for k in range(3):
        m = jnp.min(d, axis=1, keepdims=True)
        ikf = jnp.min(jnp.where(d == m, iota_f, jnp.float32(S)),
                      axis=1, keepdims=True)
        idxs.append(ikf.astype(jnp.int32))
        ws.append(1.0 / (m + 1e-8))
        if k < 2:
            d = jnp.where(iota_f == ikf, jnp.float32(jnp.inf), d)
    wcat = jnp.concatenate(ws, axis=1)
    wcat = wcat / jnp.sum(wcat, axis=1, keepdims=True)
    icat = jnp.concatenate(idxs, axis=1) + b * S
    idx_ref[0] = icat
    w_ref[0] = wcat


def _topk(xyz1, xyz2t, NT):
    B, N, _ = xyz1.shape
    S = xyz2t.shape[2]
    grid = (B, N // NT)
    return pl.pallas_call(
        functools.partial(_topk_body, S=S),
        grid=grid,
        in_specs=[
            pl.BlockSpec((1, NT, 3), lambda b, i: (b, i, 0)),
            pl.BlockSpec((1, 3, S), lambda b, i: (b, 0, 0)),
        ],
        out_specs=[
            pl.BlockSpec((1, NT, 3), lambda b, i: (b, i, 0)),
            pl.BlockSpec((1, NT, 3), lambda b, i: (b, i, 0)),
        ],
        out_shape=[
            jax.ShapeDtypeStruct((B, N, 3), jnp.int32),
            jax.ShapeDtypeStruct((B, N, 3), jnp.float32),
        ],
    )(xyz1, xyz2t)


def _sc_gather(table, idx, CH=128):
    M, = idx.shape
    R, C = table.shape
    info = plsc.get_sparse_core_info()
    NW = info.num_cores * info.num_subcores
    n_ch = M // (NW * CH)
    idx2 = idx.reshape(M // CH, CH)
    mesh = plsc.VectorSubcoreMesh(core_axis_name="c", subcore_axis_name="s")

    per_w = n_ch * CH
    idx2 = idx.reshape(NW, per_w)

    @functools.partial(
        pl.kernel,
        mesh=mesh,
        out_type=jax.ShapeDtypeStruct((M // CH, CH, C), jnp.float32),
        scratch_types=[
            pltpu.VMEM((per_w,), jnp.int32),
            pltpu.VMEM((CH, C), jnp.float32),
            pltpu.VMEM((CH, C), jnp.float32),
            pltpu.SemaphoreType.DMA,
            pltpu.SemaphoreType.DMA,
        ],
    )
    def gather_k(idx_hbm, table_hbm, out_hbm, idx_v, rows0, rows1, sem0, sem1):
        wid = lax.axis_index("c") * info.num_subcores + lax.axis_index("s")
        pltpu.sync_copy(idx_hbm.at[wid], idx_v)
        pltpu.async_copy(table_hbm.at[idx_v.at[pl.ds(0, CH)]], rows0, sem0)

        def body(jj, carry):
            for p in range(2):
                j = jj * 2 + p
                rows_cur, sem_cur = (rows0, sem0) if p == 0 else (rows1, sem1)
                rows_nxt, sem_nxt = (rows1, sem1) if p == 0 else (rows0, sem0)

                @pl.when(j + 1 < n_ch)
                def _():
                    off = pl.multiple_of((j + 1) * CH, CH)
                    pltpu.async_copy(table_hbm.at[idx_v.at[pl.ds(off, CH)]],
                                     rows_nxt, sem_nxt)

                pltpu.make_async_copy(table_hbm.at[idx_v.at[pl.ds(0, CH)]],
                                      rows_cur, sem_cur).wait()
                pltpu.sync_copy(rows_cur, out_hbm.at[wid * n_ch + j])
            return carry

        lax.fori_loop(0, n_ch // 2, body, 0)

    return gather_k(idx2, table).reshape(M, C)


def _mlp1_body(g_ref, w_ref, p1_ref, W1t_ref, b1_ref, y_ref, s_ref):
    first = (pl.program_id(0) == 0) & (pl.program_id(1) == 0)
    g = g_ref[...]
    w = w_ref[0]
    interp = (g[0, 0] * w[:, 0:1] + g[1, 0] * w[:, 1:2] + g[2, 0] * w[:, 2:3])
    x = jnp.concatenate([p1_ref[0], interp], axis=1)
    y = jnp.dot(x, W1t_ref[...], preferred_element_type=jnp.float32)
    y = y + b1_ref[...]
    y_ref[0] = y
    acc = jnp.concatenate([jnp.sum(y, axis=0, keepdims=True),
                           jnp.sum(y * y, axis=0, keepdims=True)], axis=0)

    @pl.when(first)
    def _():
        s_ref[...] = jnp.zeros_like(s_ref)

    s_ref[...] += acc


def _mlp1(g4, w3, p1t, W1t, b1row, NT):
    _, B, N, C2 = g4.shape
    C1 = p1t.shape[2]
    Co = W1t.shape[1]
    grid = (B, N // NT)
    return pl.pallas_call(
        _mlp1_body,
        grid=grid,
        in_specs=[
            pl.BlockSpec((3, 1, NT, C2), lambda b, i: (0, b, i, 0)),
            pl.BlockSpec((1, NT, 3), lambda b, i: (b, i, 0)),
            pl.BlockSpec((1, NT, C1), lambda b, i: (b, i, 0)),
            pl.BlockSpec((C1 + C2, Co), lambda b, i: (0, 0)),
            pl.BlockSpec((1, Co), lambda b, i: (0, 0)),
        ],
        out_specs=[
            pl.BlockSpec((1, NT, Co), lambda b, i: (b, i, 0)),
            pl.BlockSpec((2, Co), lambda b, i: (0, 0)),
        ],
        out_shape=[
            jax.ShapeDtypeStruct((B, N, Co), jnp.float32),
            jax.ShapeDtypeStruct((2, Co), jnp.float32),
        ],
    )(g4, w3, p1t, W1t, b1row)


def _mlp2_body(y1_ref, sc1_ref, W2t_ref, b2_ref, y2_ref, s_ref):
    first = (pl.program_id(0) == 0) & (pl.program_id(1) == 0)
    sc = sc1_ref[...]
    z = jnp.maximum(y1_ref[0] * sc[0:1, :] + sc[1:2, :], 0.0)
    y2 = jnp.dot(z, W2t_ref[...], preferred_element_type=jnp.float32)
    y2 = y2 + b2_ref[...]
    y2_ref[0] = y2
    acc = jnp.concatenate([jnp.sum(y2, axis=0, keepdims=True),
                           jnp.sum(y2 * y2, axis=0, keepdims=True)], axis=0)

    @pl.when(first)
    def _():
        s_ref[...] = jnp.zeros_like(s_ref)

    s_ref[...] += acc


def _mlp2(y1, sc1, W2t, b2row, NT):
    B, N, Ci = y1.shape
    Co = W2t.shape[1]
    grid = (B, N // NT)
    return pl.pallas_call(
        _mlp2_body,
        grid=grid,
        in_specs=[
            pl.BlockSpec((1, NT, Ci), lambda b, i: (b, i, 0)),
            pl.BlockSpec((2, Ci), lambda b, i: (0, 0)),
            pl.BlockSpec((Ci, Co), lambda b, i: (0, 0)),
            pl.BlockSpec((1, Co), lambda b, i: (0, 0)),
        ],
        out_specs=[
            pl.BlockSpec((1, NT, Co), lambda b, i: (b, i, 0)),
            pl.BlockSpec((2, Co), lambda b, i: (0, 0)),
        ],
        out_shape=[
            jax.ShapeDtypeStruct((B, N, Co), jnp.float32),
            jax.ShapeDtypeStruct((2, Co), jnp.float32),
        ],
    )(y1, sc1, W2t, b2row)


def _bn2_body(y2_ref, sc2_ref, o_ref):
    sc = sc2_ref[...]
    o_ref[0] = jnp.maximum(y2_ref[0] * sc[0:1, :] + sc[1:2, :], 0.0)


def _bn2(y2, sc2, NT):
    B, N, Co = y2.shape
    grid = (B, N // NT)
    return pl.pallas_call(
        _bn2_body,
        grid=grid,
        in_specs=[
            pl.BlockSpec((1, NT, Co), lambda b, i: (b, i, 0)),
            pl.BlockSpec((2, Co), lambda b, i: (0, 0)),
        ],
        out_specs=pl.BlockSpec((1, NT, Co), lambda b, i: (b, i, 0)),
        out_shape=jax.ShapeDtypeStruct((B, N, Co), jnp.float32),
    )(y2, sc2)


def _bn_scale_shift(sums, count, g, be):
    mean = sums[0] / count
    var = sums[1] / count - mean * mean
    scale = g * lax.rsqrt(var + 1e-5)
    shift = be - mean * scale
    return jnp.stack([scale, shift])


def kernel(xyz1, xyz2, points1, points2, W1, b1, g1, be1, W2, b2, g2, be2):
    B, N, _ = xyz1.shape
    S = xyz2.shape[1]
    C1 = points1.shape[1]
    C2 = points2.shape[1]
    NT = 512

    xyz2t = jnp.transpose(xyz2, (0, 2, 1))
    idx3, w3 = _topk(xyz1, xyz2t, NT)

    pts2 = jnp.transpose(points2, (0, 2, 1)).reshape(B * S, C2)
    idx_flat = jnp.transpose(idx3, (2, 0, 1)).reshape(-1)
    gathered = _sc_gather(pts2, idx_flat)
    g4 = gathered.reshape(3, B, N, C2)

    p1t = jnp.transpose(points1, (0, 2, 1))
    y1, s1 = _mlp1(g4, w3, p1t, jnp.transpose(W1), b1[None, :], NT)
    sc1 = _bn_scale_shift(s1, B * N, g1, be1)
    y2, s2 = _mlp2(y1, sc1, jnp.transpose(W2), b2[None, :], NT)
    sc2 = _bn_scale_shift(s2, B * N, g2, be2)
    outt = _bn2(y2, sc2, NT)
    return jnp.transpose(outt, (0, 2, 1))


# speedup vs baseline: 1.3003x; 1.0792x over previous
import functools

import jax
import jax.numpy as jnp
from jax import lax
from jax.experimental import pallas as pl
from jax.experimental.pallas import tpu as pltpu
from jax.experimental.pallas import tpu_sc as plsc


def _topk_body(x1_ref, x2t_ref, idx_ref, w_ref, *, S, b0):
    b = pl.program_id(0) + b0
    x1 = x1_ref[0]
    x2 = x2t_ref[0]
    NT = x1.shape[0]
    d = None
    for c in range(3):
        diff = x1[:, c:c + 1] - x2[c:c + 1, :]
        sq = diff * diff
        d = sq if d is None else d + sq
    iota_f = lax.broadcasted_iota(jnp.int32, (NT, S), 1).astype(jnp.float32)
    idxs, ws = [], []
    for k in range(3):
        m = jnp.min(d, axis=1, keepdims=True)
        ikf = jnp.min(jnp.where(d == m, iota_f, jnp.float32(S)),
                      axis=1, keepdims=True)
        idxs.append(ikf.astype(jnp.int32))
        ws.append(1.0 / (m + 1e-8))
        if k < 2:
            d = jnp.where(iota_f == ikf, jnp.float32(jnp.inf), d)
    wcat = jnp.concatenate(ws, axis=1)
    wcat = wcat / jnp.sum(wcat, axis=1, keepdims=True)
    icat = jnp.concatenate(idxs, axis=1) + b * S
    idx_ref[0] = icat
    w_ref[0] = wcat


def _topk(xyz1, xyz2t, NT, b0, Bh):
    _, N, _ = xyz1.shape
    S = xyz2t.shape[2]
    grid = (Bh, N // NT)
    return pl.pallas_call(
        functools.partial(_topk_body, S=S, b0=b0),
        grid=grid,
        in_specs=[
            pl.BlockSpec((1, NT, 3), lambda b, i: (b0 + b, i, 0)),
            pl.BlockSpec((1, 3, S), lambda b, i: (b0 + b, 0, 0)),
        ],
        out_specs=[
            pl.BlockSpec((1, NT, 3), lambda b, i: (b, i, 0)),
            pl.BlockSpec((1, NT, 3), lambda b, i: (b, i, 0)),
        ],
        out_shape=[
            jax.ShapeDtypeStruct((Bh, N, 3), jnp.int32),
            jax.ShapeDtypeStruct((Bh, N, 3), jnp.float32),
        ],
    )(xyz1, xyz2t)


def _sc_gather(table, idx, CH=128):
    M, = idx.shape
    R, C = table.shape
    info = plsc.get_sparse_core_info()
    NW = info.num_cores * info.num_subcores
    n_ch = M // (NW * CH)
    idx2 = idx.reshape(M // CH, CH)
    mesh = plsc.VectorSubcoreMesh(core_axis_name="c", subcore_axis_name="s")

    per_w = n_ch * CH
    idx2 = idx.reshape(NW, per_w)

    @functools.partial(
        pl.kernel,
        mesh=mesh,
        out_type=jax.ShapeDtypeStruct((M // CH, CH, C), jnp.float32),
        scratch_types=[
            pltpu.VMEM((per_w,), jnp.int32),
            pltpu.VMEM((CH, C), jnp.float32),
            pltpu.VMEM((CH, C), jnp.float32),
            pltpu.SemaphoreType.DMA,
            pltpu.SemaphoreType.DMA,
        ],
    )
    def gather_k(idx_hbm, table_hbm, out_hbm, idx_v, rows0, rows1, sem0, sem1):
        wid = lax.axis_index("c") * info.num_subcores + lax.axis_index("s")
        pltpu.sync_copy(idx_hbm.at[wid], idx_v)
        pltpu.async_copy(table_hbm.at[idx_v.at[pl.ds(0, CH)]], rows0, sem0)

        def body(jj, carry):
            for p in range(2):
                j = jj * 2 + p
                rows_cur, sem_cur = (rows0, sem0) if p == 0 else (rows1, sem1)
                rows_nxt, sem_nxt = (rows1, sem1) if p == 0 else (rows0, sem0)

                @pl.when(j + 1 < n_ch)
                def _():
                    off = pl.multiple_of((j + 1) * CH, CH)
                    pltpu.async_copy(table_hbm.at[idx_v.at[pl.ds(off, CH)]],
                                     rows_nxt, sem_nxt)

                pltpu.make_async_copy(table_hbm.at[idx_v.at[pl.ds(0, CH)]],
                                      rows_cur, sem_cur).wait()
                pltpu.sync_copy(rows_cur, out_hbm.at[wid * n_ch + j])
            return carry

        lax.fori_loop(0, n_ch // 2, body, 0)

    return gather_k(idx2, table).reshape(M, C)


def _mlp1_body(g_ref, w_ref, p1_ref, W1t_ref, b1_ref, y_ref, s_ref):
    first = (pl.program_id(0) == 0) & (pl.program_id(1) == 0)
    g = g_ref[...]
    w = w_ref[0]
    interp = (g[0, 0] * w[:, 0:1] + g[1, 0] * w[:, 1:2] + g[2, 0] * w[:, 2:3])
    x = jnp.concatenate([p1_ref[0], interp], axis=1)
    y = jnp.dot(x, W1t_ref[...], preferred_element_type=jnp.float32)
    y = y + b1_ref[...]
    y_ref[0] = y
    acc = jnp.concatenate([jnp.sum(y, axis=0, keepdims=True),
                           jnp.sum(y * y, axis=0, keepdims=True)], axis=0)

    @pl.when(first)
    def _():
        s_ref[...] = jnp.zeros_like(s_ref)

    s_ref[...] += acc


def _mlp1(g4, w3, p1t, W1t, b1row, NT, b0):
    _, Bh, N, C2 = g4.shape
    C1 = p1t.shape[2]
    Co = W1t.shape[1]
    grid = (Bh, N // NT)
    return pl.pallas_call(
        _mlp1_body,
        grid=grid,
        in_specs=[
            pl.BlockSpec((3, 1, NT, C2), lambda b, i: (0, b, i, 0)),
            pl.BlockSpec((1, NT, 3), lambda b, i: (b, i, 0)),
            pl.BlockSpec((1, NT, C1), lambda b, i: (b0 + b, i, 0)),
            pl.BlockSpec((C1 + C2, Co), lambda b, i: (0, 0)),
            pl.BlockSpec((1, Co), lambda b, i: (0, 0)),
        ],
        out_specs=[
            pl.BlockSpec((1, NT, Co), lambda b, i: (b, i, 0)),
            pl.BlockSpec((2, Co), lambda b, i: (0, 0)),
        ],
        out_shape=[
            jax.ShapeDtypeStruct((Bh, N, Co), jnp.float32),
            jax.ShapeDtypeStruct((2, Co), jnp.float32),
        ],
    )(g4, w3, p1t, W1t, b1row)


def _mlp2_body(y1_ref, sc1_ref, W2t_ref, b2_ref, y2_ref, s_ref):
    first = (pl.program_id(0) == 0) & (pl.program_id(1) == 0)
    sc = sc1_ref[...]
    z = jnp.maximum(y1_ref[0] * sc[0:1, :] + sc[1:2, :], 0.0)
    y2 = jnp.dot(z, W2t_ref[...], preferred_element_type=jnp.float32)
    y2 = y2 + b2_ref[...]
    y2_ref[0] = y2
    acc = jnp.concatenate([jnp.sum(y2, axis=0, keepdims=True),
                           jnp.sum(y2 * y2, axis=0, keepdims=True)], axis=0)

    @pl.when(first)
    def _():
        s_ref[...] = jnp.zeros_like(s_ref)

    s_ref[...] += acc


def _mlp2(y1, sc1, W2t, b2row, NT):
    B, N, Ci = y1.shape
    Co = W2t.shape[1]
    grid = (B, N // NT)
    return pl.pallas_call(
        _mlp2_body,
        grid=grid,
        in_specs=[
            pl.BlockSpec((1, NT, Ci), lambda b, i: (b, i, 0)),
            pl.BlockSpec((2, Ci), lambda b, i: (0, 0)),
            pl.BlockSpec((Ci, Co), lambda b, i: (0, 0)),
            pl.BlockSpec((1, Co), lambda b, i: (0, 0)),
        ],
        out_specs=[
            pl.BlockSpec((1, NT, Co), lambda b, i: (b, i, 0)),
            pl.BlockSpec((2, Co), lambda b, i: (0, 0)),
        ],
        out_shape=[
            jax.ShapeDtypeStruct((B, N, Co), jnp.float32),
            jax.ShapeDtypeStruct((2, Co), jnp.float32),
        ],
    )(y1, sc1, W2t, b2row)


def _bn2_body(y2_ref, sc2_ref, o_ref):
    sc = sc2_ref[...]
    z = jnp.maximum(y2_ref[0] * sc[0:1, :] + sc[1:2, :], 0.0)
    o_ref[0] = z.T


def _bn2(y2, sc2, NT):
    Bh, N, Co = y2.shape
    grid = (Bh, N // NT)
    return pl.pallas_call(
        _bn2_body,
        grid=grid,
        in_specs=[
            pl.BlockSpec((1, NT, Co), lambda b, i: (b, i, 0)),
            pl.BlockSpec((2, Co), lambda b, i: (0, 0)),
        ],
        out_specs=pl.BlockSpec((1, Co, NT), lambda b, i: (b, 0, i)),
        out_shape=jax.ShapeDtypeStruct((Bh, Co, N), jnp.float32),
    )(y2, sc2)


def _bn_scale_shift(sums, count, g, be):
    mean = sums[0] / count
    var = sums[1] / count - mean * mean
    scale = g * lax.rsqrt(var + 1e-5)
    shift = be - mean * scale
    return jnp.stack([scale, shift])


def kernel(xyz1, xyz2, points1, points2, W1, b1, g1, be1, W2, b2, g2, be2):
    B, N, _ = xyz1.shape
    S = xyz2.shape[1]
    C1 = points1.shape[1]
    C2 = points2.shape[1]
    NT = 512

    xyz2t = jnp.transpose(xyz2, (0, 2, 1))
    pts2 = jnp.transpose(points2, (0, 2, 1)).reshape(B * S, C2)
    p1t = jnp.transpose(points1, (0, 2, 1))
    W1t, W2t = jnp.transpose(W1), jnp.transpose(W2)

    Bh = B // 2
    halves = []
    for h in range(2):
        b0 = h * Bh
        idx3, w3 = _topk(xyz1, xyz2t, NT, b0, Bh)
        idx_flat = jnp.transpose(idx3, (2, 0, 1)).reshape(-1)
        gathered = _sc_gather(pts2, idx_flat)
        halves.append((gathered.reshape(3, Bh, N, C2), w3, b0))

    y1s, s1s = [], []
    for g4, w3, b0 in halves:
        y1, s1 = _mlp1(g4, w3, p1t, W1t, b1[None, :], NT, b0)
        y1s.append(y1)
        s1s.append(s1)
    sc1 = _bn_scale_shift(s1s[0] + s1s[1], B * N, g1, be1)
    y2s, s2s = [], []
    for y1 in y1s:
        y2, s2 = _mlp2(y1, sc1, W2t, b2[None, :], NT)
        y2s.append(y2)
        s2s.append(s2)
    sc2 = _bn_scale_shift(s2s[0] + s2s[1], B * N, g2, be2)
    outs = [_bn2(y2, sc2, NT) for y2 in y2s]
    return jnp.concatenate(outs, axis=0)


# speedup vs baseline: 1.3070x; 1.0051x over previous
import functools

import jax
import jax.numpy as jnp
from jax import lax
from jax.experimental import pallas as pl
from jax.experimental.pallas import tpu as pltpu
from jax.experimental.pallas import tpu_sc as plsc


def _topk_body(x1_ref, x2t_ref, idx_ref, w_ref, *, S, b0):
    b = pl.program_id(0) + b0
    x1 = x1_ref[0]
    x2 = x2t_ref[0]
    NT = x1.shape[0]
    d = None
    for c in range(3):
        diff = x1[:, c:c + 1] - x2[c:c + 1, :]
        sq = diff * diff
        d = sq if d is None else d + sq
    iota_f = lax.broadcasted_iota(jnp.int32, (NT, S), 1).astype(jnp.float32)
    idxs, ws = [], []
    for k in range(3):
        m = jnp.min(d, axis=1, keepdims=True)
        ikf = jnp.min(jnp.where(d == m, iota_f, jnp.float32(S)),
                      axis=1, keepdims=True)
        idxs.append(ikf.astype(jnp.int32))
        ws.append(1.0 / (m + 1e-8))
        if k < 2:
            d = jnp.where(iota_f == ikf, jnp.float32(jnp.inf), d)
    wcat = jnp.concatenate(ws, axis=1)
    wcat = wcat / jnp.sum(wcat, axis=1, keepdims=True)
    icat = jnp.concatenate(idxs, axis=1) + b * S
    idx_ref[0] = icat
    w_ref[0] = wcat


def _topk(xyz1, xyz2t, NT, b0, Bh):
    _, N, _ = xyz1.shape
    S = xyz2t.shape[2]
    grid = (Bh, N // NT)
    return pl.pallas_call(
        functools.partial(_topk_body, S=S, b0=b0),
        grid=grid,
        in_specs=[
            pl.BlockSpec((1, NT, 3), lambda b, i: (b0 + b, i, 0)),
            pl.BlockSpec((1, 3, S), lambda b, i: (b0 + b, 0, 0)),
        ],
        out_specs=[
            pl.BlockSpec((1, NT, 3), lambda b, i: (b, i, 0)),
            pl.BlockSpec((1, NT, 3), lambda b, i: (b, i, 0)),
        ],
        out_shape=[
            jax.ShapeDtypeStruct((Bh, N, 3), jnp.int32),
            jax.ShapeDtypeStruct((Bh, N, 3), jnp.float32),
        ],
    )(xyz1, xyz2t)


def _sc_gather(table, idx, CH=128):
    M, = idx.shape
    R, C = table.shape
    info = plsc.get_sparse_core_info()
    NW = info.num_cores * info.num_subcores
    n_ch = M // (NW * CH)
    idx2 = idx.reshape(M // CH, CH)
    mesh = plsc.VectorSubcoreMesh(core_axis_name="c", subcore_axis_name="s")

    per_w = n_ch * CH
    idx2 = idx.reshape(NW, per_w)

    @functools.partial(
        pl.kernel,
        mesh=mesh,
        out_type=jax.ShapeDtypeStruct((M // CH, CH, C), jnp.float32),
        scratch_types=[
            pltpu.VMEM((per_w,), jnp.int32),
            pltpu.VMEM((CH, C), jnp.float32),
            pltpu.VMEM((CH, C), jnp.float32),
            pltpu.SemaphoreType.DMA,
            pltpu.SemaphoreType.DMA,
        ],
    )
    def gather_k(idx_hbm, table_hbm, out_hbm, idx_v, rows0, rows1, sem0, sem1):
        wid = lax.axis_index("c") * info.num_subcores + lax.axis_index("s")
        pltpu.sync_copy(idx_hbm.at[wid], idx_v)
        pltpu.async_copy(table_hbm.at[idx_v.at[pl.ds(0, CH)]], rows0, sem0)

        def body(jj, carry):
            for p in range(2):
                j = jj * 2 + p
                rows_cur, sem_cur = (rows0, sem0) if p == 0 else (rows1, sem1)
                rows_nxt, sem_nxt = (rows1, sem1) if p == 0 else (rows0, sem0)

                @pl.when(j + 1 < n_ch)
                def _():
                    off = pl.multiple_of((j + 1) * CH, CH)
                    pltpu.async_copy(table_hbm.at[idx_v.at[pl.ds(off, CH)]],
                                     rows_nxt, sem_nxt)

                pltpu.make_async_copy(table_hbm.at[idx_v.at[pl.ds(0, CH)]],
                                      rows_cur, sem_cur).wait()
                pltpu.sync_copy(rows_cur, out_hbm.at[wid * n_ch + j])
            return carry

        lax.fori_loop(0, n_ch // 2, body, 0)

    return gather_k(idx2, table).reshape(M, C)


def _mlp1_body(g_ref, w_ref, p1_ref, W1t_ref, b1_ref, y_ref, s_ref):
    first = (pl.program_id(0) == 0) & (pl.program_id(1) == 0)
    g = g_ref[...]
    w = w_ref[0]
    interp = (g[0, 0] * w[:, 0:1] + g[1, 0] * w[:, 1:2] + g[2, 0] * w[:, 2:3])
    x = jnp.concatenate([p1_ref[0], interp], axis=1)
    y = jnp.dot(x, W1t_ref[...], preferred_element_type=jnp.float32)
    y = y + b1_ref[...]
    y_ref[0] = y
    acc = jnp.concatenate([jnp.sum(y, axis=0, keepdims=True),
                           jnp.sum(y * y, axis=0, keepdims=True)], axis=0)

    @pl.when(first)
    def _():
        s_ref[...] = jnp.zeros_like(s_ref)

    s_ref[...] += acc


def _mlp1(g4, w3, p1t, W1t, b1row, NT, b0):
    _, Bh, N, C2 = g4.shape
    C1 = p1t.shape[2]
    Co = W1t.shape[1]
    grid = (Bh, N // NT)
    return pl.pallas_call(
        _mlp1_body,
        grid=grid,
        in_specs=[
            pl.BlockSpec((3, 1, NT, C2), lambda b, i: (0, b, i, 0)),
            pl.BlockSpec((1, NT, 3), lambda b, i: (b, i, 0)),
            pl.BlockSpec((1, NT, C1), lambda b, i: (b0 + b, i, 0)),
            pl.BlockSpec((C1 + C2, Co), lambda b, i: (0, 0)),
            pl.BlockSpec((1, Co), lambda b, i: (0, 0)),
        ],
        out_specs=[
            pl.BlockSpec((1, NT, Co), lambda b, i: (b, i, 0)),
            pl.BlockSpec((2, Co), lambda b, i: (0, 0)),
        ],
        out_shape=[
            jax.ShapeDtypeStruct((Bh, N, Co), jnp.float32),
            jax.ShapeDtypeStruct((2, Co), jnp.float32),
        ],
    )(g4, w3, p1t, W1t, b1row)


def _scale_shift(sa, sb, gb, count):
    s = sa + sb
    mean = s[0:1, :] / count
    var = s[1:2, :] / count - mean * mean
    scale = gb[0:1, :] * lax.rsqrt(var + 1e-5)
    shift = gb[1:2, :] - mean * scale
    return scale, shift


def _mlp2_body(y1_ref, sa_ref, sb_ref, gb_ref, W2t_ref, b2_ref, y2_ref, s_ref,
               *, count):
    first = (pl.program_id(0) == 0) & (pl.program_id(1) == 0)
    scale, shift = _scale_shift(sa_ref[...], sb_ref[...], gb_ref[...], count)
    z = jnp.maximum(y1_ref[0] * scale + shift, 0.0)
    y2 = jnp.dot(z, W2t_ref[...], preferred_element_type=jnp.float32)
    y2 = y2 + b2_ref[...]
    y2_ref[0] = y2
    acc = jnp.concatenate([jnp.sum(y2, axis=0, keepdims=True),
                           jnp.sum(y2 * y2, axis=0, keepdims=True)], axis=0)

    @pl.when(first)
    def _():
        s_ref[...] = jnp.zeros_like(s_ref)

    s_ref[...] += acc


def _mlp2(y1, sa, sb, gb, W2t, b2row, NT, count):
    B, N, Ci = y1.shape
    Co = W2t.shape[1]
    grid = (B, N // NT)
    return pl.pallas_call(
        functools.partial(_mlp2_body, count=count),
        grid=grid,
        in_specs=[
            pl.BlockSpec((1, NT, Ci), lambda b, i: (b, i, 0)),
            pl.BlockSpec((2, Ci), lambda b, i: (0, 0)),
            pl.BlockSpec((2, Ci), lambda b, i: (0, 0)),
            pl.BlockSpec((2, Ci), lambda b, i: (0, 0)),
            pl.BlockSpec((Ci, Co), lambda b, i: (0, 0)),
            pl.BlockSpec((1, Co), lambda b, i: (0, 0)),
        ],
        out_specs=[
            pl.BlockSpec((1, NT, Co), lambda b, i: (b, i, 0)),
            pl.BlockSpec((2, Co), lambda b, i: (0, 0)),
        ],
        out_shape=[
            jax.ShapeDtypeStruct((B, N, Co), jnp.float32),
            jax.ShapeDtypeStruct((2, Co), jnp.float32),
        ],
    )(y1, sa, sb, gb, W2t, b2row)


def _bn2_body(y2_ref, sa_ref, sb_ref, gb_ref, o_ref, *, count):
    scale, shift = _scale_shift(sa_ref[...], sb_ref[...], gb_ref[...], count)
    z = jnp.maximum(y2_ref[0] * scale + shift, 0.0)
    o_ref[0] = z.T


def _bn2(y2, sa, sb, gb, NT, count):
    Bh, N, Co = y2.shape
    grid = (Bh, N // NT)
    return pl.pallas_call(
        functools.partial(_bn2_body, count=count),
        grid=grid,
        in_specs=[
            pl.BlockSpec((1, NT, Co), lambda b, i: (b, i, 0)),
            pl.BlockSpec((2, Co), lambda b, i: (0, 0)),
            pl.BlockSpec((2, Co), lambda b, i: (0, 0)),
            pl.BlockSpec((2, Co), lambda b, i: (0, 0)),
        ],
        out_specs=pl.BlockSpec((1, Co, NT), lambda b, i: (b, 0, i)),
        out_shape=jax.ShapeDtypeStruct((Bh, Co, N), jnp.float32),
    )(y2, sa, sb, gb)


def kernel(xyz1, xyz2, points1, points2, W1, b1, g1, be1, W2, b2, g2, be2):
    B, N, _ = xyz1.shape
    S = xyz2.shape[1]
    C1 = points1.shape[1]
    C2 = points2.shape[1]
    NT = 512

    xyz2t = jnp.transpose(xyz2, (0, 2, 1))
    pts2 = jnp.transpose(points2, (0, 2, 1)).reshape(B * S, C2)
    p1t = jnp.transpose(points1, (0, 2, 1))
    W1t, W2t = jnp.transpose(W1), jnp.transpose(W2)

    Bh = B // 2
    halves = []
    for h in range(2):
        b0 = h * Bh
        idx3, w3 = _topk(xyz1, xyz2t, NT, b0, Bh)
        idx_flat = jnp.transpose(idx3, (2, 0, 1)).reshape(-1)
        gathered = _sc_gather(pts2, idx_flat)
        halves.append((gathered.reshape(3, Bh, N, C2), w3, b0))

    cnt = float(B * N)
    gb1 = jnp.stack([g1, be1])
    gb2 = jnp.stack([g2, be2])
    y1s, s1s = [], []
    for g4, w3, b0 in halves:
        y1, s1 = _mlp1(g4, w3, p1t, W1t, b1[None, :], NT, b0)
        y1s.append(y1)
        s1s.append(s1)
    y2s, s2s = [], []
    for y1 in y1s:
        y2, s2 = _mlp2(y1, s1s[0], s1s[1], gb1, W2t, b2[None, :], NT, cnt)
        y2s.append(y2)
        s2s.append(s2)
    outs = [_bn2(y2, s2s[0], s2s[1], gb2, NT, cnt) for y2 in y2s]
    return jnp.concatenate(outs, axis=0)
